# Initial kernel scaffold; baseline (speedup 1.0000x reference)
import jax
import jax.numpy as jnp
from jax.experimental import pallas as pl


def kernel(features, labels, bi_adj, adjacency_mask, Wd, bd, Wf, bf, Ww, bw, Wa, ba, Wl, bl, edge_index):
    raise NotImplementedError("write your pallas kernel here")


# speedup vs baseline: 62.0528x; 62.0528x over previous
import functools
import jax
import jax.numpy as jnp
from jax import lax
from jax.experimental import pallas as pl
from jax.experimental.pallas import tpu as pltpu
from jax.experimental.pallas import tpu_sc as plsc

N = 4096
E = 131072
IN_D = 128
OUT_D = 64
HEAD = 4
HD = 64
DH = OUT_D * HEAD

NW = 32
EPW = E // NW
BE = 64
NB = EPW // BE
GPB = BE // 16

_f32 = jnp.float32
_i32 = jnp.int32


def _pre_body(f_ref, Wd_ref, bd_ref, Wf_ref, bf_ref, Ww_ref, bw_ref, Wa_ref,
              ba_ref, h_ref, nt_ref, bv_ref):
    f = f_ref[...]
    Wd = Wd_ref[...]
    wf1 = Wf_ref[0:DH, :]
    wf2 = Wf_ref[DH:2 * DH, :]
    wf3 = Wf_ref[2 * DH:3 * DH, :]
    w13 = wf1 + wf3
    w23 = wf2 - wf3
    va = Wd @ w13
    vb = Wd @ w23
    ca = (bd_ref[...] @ w13)[0, 0] + bf_ref[0, 0]
    cb = (bd_ref[...] @ w23)[0, 0]
    a = (f @ va)[:, 0] + ca
    b = (f @ vb)[:, 0] + cb
    h = f @ Ww_ref[...] + bw_ref[...]
    h_ref[...] = h

    wa1 = Wa_ref[0:HD, :]
    wa2 = Wa_ref[HD:2 * HD, :]
    rows = [a[None, :], b[None, :]]
    ps = []
    qs = []
    for hh in range(HEAD):
        blk = h[:, hh * HD:(hh + 1) * HD]
        ps.append((blk @ wa1)[:, 0])
        qs.append((blk @ wa2)[:, 0] + ba_ref[0, 0])
    for hh in range(HEAD):
        rows.append(ps[hh][None, :])
    for hh in range(HEAD):
        rows.append(qs[hh][None, :])
    nt_ref[...] = jnp.concatenate(rows, axis=0)
    brows = []
    for hh in range(HEAD):
        Bh = jnp.max(jnp.abs(ps[hh])) + jnp.max(jnp.abs(qs[hh]))
        brows.append(jnp.broadcast_to(Bh, (1, 128)))
    brows.append(jnp.zeros((4, 128), _f32))
    bv_ref[...] = jnp.concatenate(brows, axis=0)


def _rden_body(parts_ref, rd_ref):
    s = parts_ref[0, :, 0:HEAD] + parts_ref[1, :, 0:HEAD]
    rd_ref[...] = 1.0 / (s + 1e-16)


def _post_body(parts_ref, Wl_ref, bl_ref, o_ref):
    o = parts_ref[0] + parts_ref[1]
    o_ref[...] = o @ Wl_ref[...] + bl_ref[...]


def _lp_body(bi_ref, mask_ref, lab_ref, y_ref):
    t = bi_ref[...] * mask_ref[...]
    den = jnp.maximum(jnp.sum(jnp.abs(t), axis=1, keepdims=True), 1e-12)
    y_ref[...] = (t @ lab_ref[...]) / den


def _splat_i(v):
    return jnp.full((16,), v, _i32)


def _edge_scalars(nt_v, srcg, dstg, bvecs):
    a_s = plsc.load_gather(nt_v, [_splat_i(0), srcg])
    b_d = plsc.load_gather(nt_v, [_splat_i(1), dstg])
    x = a_s + b_d
    t = jnp.exp(-2.0 * jnp.abs(x))
    score = jnp.sign(x) * (1.0 - t) / (1.0 + t)
    exs = []
    for hh in range(HEAD):
        p_s = plsc.load_gather(nt_v, [_splat_i(2 + hh), srcg])
        q_d = plsc.load_gather(nt_v, [_splat_i(6 + hh), dstg])
        z = score * p_s + q_d
        alpha = jnp.maximum(z, 0.01 * z)
        exs.append(jnp.exp(alpha - bvecs[hh]))
    return score, exs


_MESH = plsc.VectorSubcoreMesh(core_axis_name="c", subcore_axis_name="s",
                               num_cores=2, num_subcores=16)


@functools.partial(
    pl.kernel,
    out_type=jax.ShapeDtypeStruct((2 * N, 16), _f32),
    mesh=_MESH,
    compiler_params=pltpu.CompilerParams(use_tc_tiling_on_sc=False, needs_layout_passes=False),
    scratch_types=[
        pltpu.VMEM((10, N), _f32),
        pltpu.VMEM((8, 128), _f32),
        pltpu.VMEM((EPW,), _i32),
        pltpu.VMEM((EPW,), _i32),
        pltpu.VMEM((8, BE), _i32),
        pltpu.VMEM((BE, 16), _f32),
        pltpu.VMEM_SHARED((N, 16), _f32),
    ],
)
def _sc_sden(nt_hbm, bv_hbm, src_hbm, dst_hbm, out_hbm,
             nt_v, bv_v, src_v, dst_v, didx_v, exb_v, sden_sh):
    cid = lax.axis_index("c")
    sid = lax.axis_index("s")
    wid = cid * 16 + sid

    pltpu.sync_copy(nt_hbm, nt_v)
    pltpu.sync_copy(bv_hbm, bv_v)
    pltpu.sync_copy(src_hbm.at[pl.ds(wid * EPW, EPW)], src_v)
    pltpu.sync_copy(dst_hbm.at[pl.ds(wid * EPW, EPW)], dst_v)

    zero = jnp.zeros((16,), _f32)
    for r in range(BE):
        exb_v[r, :] = zero
    for tt in range(4):
        pltpu.sync_copy(exb_v, sden_sh.at[pl.ds(sid * 256 + tt * BE, BE)])
    plsc.subcore_barrier()

    iota = lax.iota(_i32, 16)
    bvecs = [bv_v[hh, pl.ds(0, 16)] for hh in range(HEAD)]

    def batch(j, carry):
        for k in range(GPB):
            srcg = src_v[pl.ds(j * BE + k * 16, 16)]
            dstg = dst_v[pl.ds(j * BE + k * 16, 16)]
            didx_v[0, pl.ds(k * 16, 16)] = dstg
            _, exs = _edge_scalars(nt_v, srcg, dstg, bvecs)
            for hh in range(HEAD):
                plsc.store_scatter(exb_v, [iota + k * 16, _splat_i(hh)],
                                   exs[hh])
        pltpu.sync_copy(exb_v, sden_sh.at[didx_v.at[0]], add=True)
        return carry

    lax.fori_loop(0, NB, batch, 0)
    plsc.subcore_barrier()
    pltpu.sync_copy(sden_sh.at[pl.ds(sid * 256, 256)],
                    out_hbm.at[pl.ds(cid * N + sid * 256, 256)])


@functools.partial(
    pl.kernel,
    out_type=jax.ShapeDtypeStruct((2 * N, HD), _f32),
    mesh=_MESH,
    compiler_params=pltpu.CompilerParams(use_tc_tiling_on_sc=False, needs_layout_passes=False),
    scratch_types=[
        pltpu.VMEM((10, N), _f32),
        pltpu.VMEM((8, 128), _f32),
        pltpu.VMEM((N, HEAD), _f32),
        pltpu.VMEM((EPW,), _i32),
        pltpu.VMEM((EPW,), _i32),
        pltpu.VMEM((8, BE), _i32),
        pltpu.VMEM((BE, DH), _f32),
        pltpu.VMEM((BE, HD), _f32),
        pltpu.VMEM_SHARED((N, HD), _f32),
        pltpu.SemaphoreType.DMA,
    ],
)
def _sc_msg(nt_hbm, bv_hbm, rd_hbm, h_hbm, src_hbm, dst_hbm,
            out_hbm, nt_v, bv_v, rd_v, src_v, dst_v, didx_v, rows_v,
            msg_v, out_sh, sem):
    cid = lax.axis_index("c")
    sid = lax.axis_index("s")
    wid = cid * 16 + sid

    pltpu.sync_copy(nt_hbm, nt_v)
    pltpu.sync_copy(bv_hbm, bv_v)
    pltpu.sync_copy(rd_hbm, rd_v)
    pltpu.sync_copy(src_hbm.at[pl.ds(wid * EPW, EPW)], src_v)
    pltpu.sync_copy(dst_hbm.at[pl.ds(wid * EPW, EPW)], dst_v)

    zero = jnp.zeros((16,), _f32)
    for r in range(BE):
        for s4 in range(HD // 16):
            msg_v[r, pl.ds(s4 * 16, 16)] = zero
    for tt in range(4):
        pltpu.sync_copy(msg_v, out_sh.at[pl.ds(sid * 256 + tt * BE, BE)])
    plsc.subcore_barrier()

    bvecs = [bv_v[hh, pl.ds(0, 16)] for hh in range(HEAD)]

    def batch(j, carry):
        cp = pltpu.async_copy(h_hbm.at[src_v.at[pl.ds(j * BE, BE)]],
                              rows_v, sem)
        cp.wait()
        for k in range(GPB):
            srcg = src_v[pl.ds(j * BE + k * 16, 16)]
            dstg = dst_v[pl.ds(j * BE + k * 16, 16)]
            didx_v[0, pl.ds(k * 16, 16)] = dstg
            score, exs = _edge_scalars(nt_v, srcg, dstg, bvecs)
            cs = []
            for hh in range(HEAD):
                rd_d = plsc.load_gather(rd_v, [dstg, _splat_i(hh)])
                cs.append(exs[hh] * rd_d * score)
            for e in range(16):
                r = k * 16 + e
                c0 = cs[0][e]
                c1 = cs[1][e]
                c2 = cs[2][e]
                c3 = cs[3][e]
                for s4 in range(HD // 16):
                    m = (c0 * rows_v[r, pl.ds(s4 * 16, 16)]
                         + c1 * rows_v[r, pl.ds(HD + s4 * 16, 16)]
                         + c2 * rows_v[r, pl.ds(2 * HD + s4 * 16, 16)]
                         + c3 * rows_v[r, pl.ds(3 * HD + s4 * 16, 16)])
                    msg_v[r, pl.ds(s4 * 16, 16)] = m
        pltpu.sync_copy(msg_v, out_sh.at[didx_v.at[0]], add=True)
        return carry

    lax.fori_loop(0, NB, batch, 0)
    plsc.subcore_barrier()
    pltpu.sync_copy(out_sh.at[pl.ds(sid * 256, 256)],
                    out_hbm.at[pl.ds(cid * N + sid * 256, 256)])


def kernel(features, labels, bi_adj, adjacency_mask, Wd, bd, Wf, bf, Ww, bw,
           Wa, ba, Wl, bl, edge_index):
    src = edge_index[0]
    dst = edge_index[1]

    h, nt, bv = pl.pallas_call(
        _pre_body,
        out_shape=[jax.ShapeDtypeStruct((N, DH), _f32),
                   jax.ShapeDtypeStruct((10, N), _f32),
                   jax.ShapeDtypeStruct((8, 128), _f32)],
    )(features, Wd, bd.reshape(1, DH), Wf, bf.reshape(1, 1), Ww,
      bw.reshape(1, DH), Wa, ba.reshape(1, 1))

    sden_parts = _sc_sden(nt, bv, src, dst).reshape(2, N, 16)

    rden = pl.pallas_call(
        _rden_body,
        out_shape=jax.ShapeDtypeStruct((N, HEAD), _f32),
    )(sden_parts)

    out_parts = _sc_msg(nt, bv, rden, h, src, dst).reshape(2, N, HD)

    h_out = pl.pallas_call(
        _post_body,
        out_shape=jax.ShapeDtypeStruct((N, OUT_D), _f32),
    )(out_parts, Wl, bl.reshape(1, OUT_D))

    lab_pad = jnp.pad(labels, ((0, 0), (0, 126)))
    RB = 256
    y_pad = pl.pallas_call(
        _lp_body,
        grid=(N // RB,),
        in_specs=[pl.BlockSpec((RB, N), lambda i: (i, 0)),
                  pl.BlockSpec((RB, N), lambda i: (i, 0)),
                  pl.BlockSpec((N, 128), lambda i: (0, 0))],
        out_specs=pl.BlockSpec((RB, 128), lambda i: (i, 0)),
        out_shape=jax.ShapeDtypeStruct((N, 128), _f32),
    )(bi_adj, adjacency_mask, lab_pad)
    y_hat = y_pad[:, :2]

    return h_out, y_hat


# speedup vs baseline: 73.6976x; 1.1877x over previous
import functools
import jax
import jax.numpy as jnp
from jax import lax
from jax.experimental import pallas as pl
from jax.experimental.pallas import tpu as pltpu
from jax.experimental.pallas import tpu_sc as plsc

N = 4096
E = 131072
IN_D = 128
OUT_D = 64
HEAD = 4
HD = 64
DH = OUT_D * HEAD

NW = 32
EPW = E // NW
BE = 32
NB = EPW // BE
GPB = BE // 16

_f32 = jnp.float32
_i32 = jnp.int32


def _pre_body(f_ref, Wd_ref, bd_ref, Wf_ref, bf_ref, Ww_ref, bw_ref, Wa_ref,
              ba_ref, h_ref, nt_ref, bv_ref):
    f = f_ref[...]
    Wd = Wd_ref[...]
    wf1 = Wf_ref[0:DH, :]
    wf2 = Wf_ref[DH:2 * DH, :]
    wf3 = Wf_ref[2 * DH:3 * DH, :]
    w13 = wf1 + wf3
    w23 = wf2 - wf3
    va = Wd @ w13
    vb = Wd @ w23
    ca = (bd_ref[...] @ w13)[0, 0] + bf_ref[0, 0]
    cb = (bd_ref[...] @ w23)[0, 0]
    a = (f @ va)[:, 0] + ca
    b = (f @ vb)[:, 0] + cb
    h = f @ Ww_ref[...] + bw_ref[...]
    h_ref[...] = h

    wa1 = Wa_ref[0:HD, :]
    wa2 = Wa_ref[HD:2 * HD, :]
    rows = [a[None, :], b[None, :]]
    ps = []
    qs = []
    for hh in range(HEAD):
        blk = h[:, hh * HD:(hh + 1) * HD]
        ps.append((blk @ wa1)[:, 0])
        qs.append((blk @ wa2)[:, 0] + ba_ref[0, 0])
    for hh in range(HEAD):
        rows.append(ps[hh][None, :])
    for hh in range(HEAD):
        rows.append(qs[hh][None, :])
    nt_ref[...] = jnp.concatenate(rows, axis=0)
    brows = []
    for hh in range(HEAD):
        Bh = jnp.max(jnp.abs(ps[hh])) + jnp.max(jnp.abs(qs[hh]))
        brows.append(jnp.broadcast_to(Bh, (1, 128)))
    brows.append(jnp.zeros((4, 128), _f32))
    bv_ref[...] = jnp.concatenate(brows, axis=0)


def _rden_body(parts_ref, rd_ref):
    s = parts_ref[0, :, 0:HEAD] + parts_ref[1, :, 0:HEAD]
    rd_ref[...] = 1.0 / (s + 1e-16)


def _post_body(parts_ref, Wl_ref, bl_ref, o_ref):
    o = parts_ref[0] + parts_ref[1]
    o_ref[...] = o @ Wl_ref[...] + bl_ref[...]


def _lp_body(bi_ref, mask_ref, lab_ref, y_ref):
    t = bi_ref[...] * mask_ref[...]
    den = jnp.maximum(jnp.sum(jnp.abs(t), axis=1, keepdims=True), 1e-12)
    y_ref[...] = (t @ lab_ref[...]) / den


def _splat_i(v):
    return jnp.full((16,), v, _i32)


def _edge_scalars(nt_v, srcg, dstg, bvecs):
    a_s = plsc.load_gather(nt_v, [_splat_i(0), srcg])
    b_d = plsc.load_gather(nt_v, [_splat_i(1), dstg])
    x = a_s + b_d
    t = jnp.exp(-2.0 * jnp.abs(x))
    score = jnp.sign(x) * (1.0 - t) / (1.0 + t)
    exs = []
    for hh in range(HEAD):
        p_s = plsc.load_gather(nt_v, [_splat_i(2 + hh), srcg])
        q_d = plsc.load_gather(nt_v, [_splat_i(6 + hh), dstg])
        z = score * p_s + q_d
        alpha = jnp.maximum(z, 0.01 * z)
        exs.append(jnp.exp(alpha - bvecs[hh]))
    return score, exs


_MESH = plsc.VectorSubcoreMesh(core_axis_name="c", subcore_axis_name="s",
                               num_cores=2, num_subcores=16)


@functools.partial(
    pl.kernel,
    out_type=jax.ShapeDtypeStruct((2 * N, 16), _f32),
    mesh=_MESH,
    compiler_params=pltpu.CompilerParams(use_tc_tiling_on_sc=False, needs_layout_passes=False),
    scratch_types=[
        pltpu.VMEM((10, N), _f32),
        pltpu.VMEM((8, 128), _f32),
        pltpu.VMEM((EPW,), _i32),
        pltpu.VMEM((EPW,), _i32),
        pltpu.VMEM((8, BE), _i32),
        pltpu.VMEM((BE, 16), _f32),
        pltpu.VMEM_SHARED((N, 16), _f32),
    ],
)
def _sc_sden(nt_hbm, bv_hbm, src_hbm, dst_hbm, out_hbm,
             nt_v, bv_v, src_v, dst_v, didx_v, exb_v, sden_sh):
    cid = lax.axis_index("c")
    sid = lax.axis_index("s")
    wid = cid * 16 + sid

    pltpu.sync_copy(nt_hbm, nt_v)
    pltpu.sync_copy(bv_hbm, bv_v)
    pltpu.sync_copy(src_hbm.at[pl.ds(wid * EPW, EPW)], src_v)
    pltpu.sync_copy(dst_hbm.at[pl.ds(wid * EPW, EPW)], dst_v)

    zero = jnp.zeros((16,), _f32)
    for r in range(BE):
        exb_v[r, :] = zero
    for tt in range(256 // BE):
        pltpu.sync_copy(exb_v, sden_sh.at[pl.ds(sid * 256 + tt * BE, BE)])
    plsc.subcore_barrier()

    iota = lax.iota(_i32, 16)
    bvecs = [bv_v[hh, pl.ds(0, 16)] for hh in range(HEAD)]

    def batch(j, carry):
        for k in range(GPB):
            srcg = src_v[pl.ds(j * BE + k * 16, 16)]
            dstg = dst_v[pl.ds(j * BE + k * 16, 16)]
            didx_v[0, pl.ds(k * 16, 16)] = dstg
            _, exs = _edge_scalars(nt_v, srcg, dstg, bvecs)
            for hh in range(HEAD):
                plsc.store_scatter(exb_v, [iota + k * 16, _splat_i(hh)],
                                   exs[hh])
        pltpu.sync_copy(exb_v, sden_sh.at[didx_v.at[0]], add=True)
        return carry

    lax.fori_loop(0, NB, batch, 0)
    plsc.subcore_barrier()
    pltpu.sync_copy(sden_sh.at[pl.ds(sid * 256, 256)],
                    out_hbm.at[pl.ds(cid * N + sid * 256, 256)])


@functools.partial(
    pl.kernel,
    out_type=jax.ShapeDtypeStruct((2 * N, HD), _f32),
    mesh=_MESH,
    compiler_params=pltpu.CompilerParams(use_tc_tiling_on_sc=False, needs_layout_passes=False),
    scratch_types=[
        pltpu.VMEM((10, N), _f32),
        pltpu.VMEM((8, 128), _f32),
        pltpu.VMEM((N, HEAD), _f32),
        pltpu.VMEM((EPW,), _i32),
        pltpu.VMEM((EPW,), _i32),
        pltpu.VMEM((8, BE), _i32),
        pltpu.VMEM((BE, DH), _f32),
        pltpu.VMEM((BE, DH), _f32),
        pltpu.VMEM((BE, HD), _f32),
        pltpu.VMEM_SHARED((N, HD), _f32),
        pltpu.SemaphoreType.DMA,
        pltpu.SemaphoreType.DMA,
    ],
)
def _sc_msg(nt_hbm, bv_hbm, rd_hbm, h_hbm, src_hbm, dst_hbm,
            out_hbm, nt_v, bv_v, rd_v, src_v, dst_v, didx_v, rows0_v,
            rows1_v, msg_v, out_sh, sem0, sem1):
    cid = lax.axis_index("c")
    sid = lax.axis_index("s")
    wid = cid * 16 + sid

    pltpu.sync_copy(nt_hbm, nt_v)
    pltpu.sync_copy(bv_hbm, bv_v)
    pltpu.sync_copy(rd_hbm, rd_v)
    pltpu.sync_copy(src_hbm.at[pl.ds(wid * EPW, EPW)], src_v)
    pltpu.sync_copy(dst_hbm.at[pl.ds(wid * EPW, EPW)], dst_v)

    zero = jnp.zeros((16,), _f32)
    for r in range(BE):
        for s4 in range(HD // 16):
            msg_v[r, pl.ds(s4 * 16, 16)] = zero
    for tt in range(256 // BE):
        pltpu.sync_copy(msg_v, out_sh.at[pl.ds(sid * 256 + tt * BE, BE)])
    plsc.subcore_barrier()

    bvecs = [bv_v[hh, pl.ds(0, 16)] for hh in range(HEAD)]

    def compute_batch(j, rows_v):
        for k in range(GPB):
            srcg = src_v[pl.ds(j * BE + k * 16, 16)]
            dstg = dst_v[pl.ds(j * BE + k * 16, 16)]
            didx_v[0, pl.ds(k * 16, 16)] = dstg
            score, exs = _edge_scalars(nt_v, srcg, dstg, bvecs)
            cs = []
            for hh in range(HEAD):
                rd_d = plsc.load_gather(rd_v, [dstg, _splat_i(hh)])
                cs.append(exs[hh] * rd_d * score)
            for e in range(16):
                r = k * 16 + e
                c0 = cs[0][e]
                c1 = cs[1][e]
                c2 = cs[2][e]
                c3 = cs[3][e]
                for s4 in range(HD // 16):
                    m = (c0 * rows_v[r, pl.ds(s4 * 16, 16)]
                         + c1 * rows_v[r, pl.ds(HD + s4 * 16, 16)]
                         + c2 * rows_v[r, pl.ds(2 * HD + s4 * 16, 16)]
                         + c3 * rows_v[r, pl.ds(3 * HD + s4 * 16, 16)])
                    msg_v[r, pl.ds(s4 * 16, 16)] = m
        pltpu.sync_copy(msg_v, out_sh.at[didx_v.at[0]], add=True)

    def gather_rows(j):
        return h_hbm.at[src_v.at[pl.ds(j * BE, BE)]]

    pltpu.async_copy(gather_rows(0), rows0_v, sem0)

    def pair(i, carry):
        j0 = 2 * i
        j1 = j0 + 1
        pltpu.async_copy(gather_rows(j1), rows1_v, sem1)
        pltpu.make_async_copy(gather_rows(j0), rows0_v, sem0).wait()
        compute_batch(j0, rows0_v)
        jn = jnp.minimum(j0 + 2, NB - 2)
        pltpu.async_copy(gather_rows(jn), rows0_v, sem0)
        pltpu.make_async_copy(gather_rows(j1), rows1_v, sem1).wait()
        compute_batch(j1, rows1_v)
        return carry

    lax.fori_loop(0, NB // 2, pair, 0)
    pltpu.make_async_copy(gather_rows(0), rows0_v, sem0).wait()
    plsc.subcore_barrier()
    pltpu.sync_copy(out_sh.at[pl.ds(sid * 256, 256)],
                    out_hbm.at[pl.ds(cid * N + sid * 256, 256)])


def kernel(features, labels, bi_adj, adjacency_mask, Wd, bd, Wf, bf, Ww, bw,
           Wa, ba, Wl, bl, edge_index):
    src = edge_index[0]
    dst = edge_index[1]

    h, nt, bv = pl.pallas_call(
        _pre_body,
        out_shape=[jax.ShapeDtypeStruct((N, DH), _f32),
                   jax.ShapeDtypeStruct((10, N), _f32),
                   jax.ShapeDtypeStruct((8, 128), _f32)],
    )(features, Wd, bd.reshape(1, DH), Wf, bf.reshape(1, 1), Ww,
      bw.reshape(1, DH), Wa, ba.reshape(1, 1))

    sden_parts = _sc_sden(nt, bv, src, dst).reshape(2, N, 16)

    rden = pl.pallas_call(
        _rden_body,
        out_shape=jax.ShapeDtypeStruct((N, HEAD), _f32),
    )(sden_parts)

    out_parts = _sc_msg(nt, bv, rden, h, src, dst).reshape(2, N, HD)

    h_out = pl.pallas_call(
        _post_body,
        out_shape=jax.ShapeDtypeStruct((N, OUT_D), _f32),
    )(out_parts, Wl, bl.reshape(1, OUT_D))

    lab_pad = jnp.pad(labels, ((0, 0), (0, 126)))
    RB = 256
    y_pad = pl.pallas_call(
        _lp_body,
        grid=(N // RB,),
        in_specs=[pl.BlockSpec((RB, N), lambda i: (i, 0)),
                  pl.BlockSpec((RB, N), lambda i: (i, 0)),
                  pl.BlockSpec((N, 128), lambda i: (0, 0))],
        out_specs=pl.BlockSpec((RB, 128), lambda i: (i, 0)),
        out_shape=jax.ShapeDtypeStruct((N, 128), _f32),
    )(bi_adj, adjacency_mask, lab_pad)
    y_hat = y_pad[:, :2]

    return h_out, y_hat


# speedup vs baseline: 77.9092x; 1.0571x over previous
import functools
import jax
import jax.numpy as jnp
from jax import lax
from jax.experimental import pallas as pl
from jax.experimental.pallas import tpu as pltpu
from jax.experimental.pallas import tpu_sc as plsc

N = 4096
E = 131072
IN_D = 128
OUT_D = 64
HEAD = 4
HD = 64
DH = OUT_D * HEAD

NW = 32
EPW = E // NW
BE = 32
NB = EPW // BE
GPB = BE // 16
SB = 2 * BE
BE1 = 128
NB1 = EPW // BE1
GPB1 = BE1 // 16

_f32 = jnp.float32
_i32 = jnp.int32


def _pre1_body(f_ref, Wd_ref, bd_ref, Wf_ref, bf_ref, Ww_ref, bw_ref, Wa_ref,
               ba_ref, nt_ref, bv_ref):
    f = f_ref[...]
    Wd = Wd_ref[...]
    Ww = Ww_ref[...]
    bw = bw_ref[...]
    wf1 = Wf_ref[0:DH, :]
    wf2 = Wf_ref[DH:2 * DH, :]
    wf3 = Wf_ref[2 * DH:3 * DH, :]
    w13 = wf1 + wf3
    w23 = wf2 - wf3
    wa1 = Wa_ref[0:HD, :]
    wa2 = Wa_ref[HD:2 * HD, :]

    cols = [Wd @ w13, Wd @ w23]
    consts = [(bd_ref[...] @ w13)[0, 0] + bf_ref[0, 0],
              (bd_ref[...] @ w23)[0, 0]]
    for hh in range(HEAD):
        blk = Ww[:, hh * HD:(hh + 1) * HD]
        bwb = bw[:, hh * HD:(hh + 1) * HD]
        cols.append(blk @ wa1)
        consts.append((bwb @ wa1)[0, 0])
    for hh in range(HEAD):
        blk = Ww[:, hh * HD:(hh + 1) * HD]
        bwb = bw[:, hh * HD:(hh + 1) * HD]
        cols.append(blk @ wa2)
        consts.append((bwb @ wa2)[0, 0] + ba_ref[0, 0])

    V = jnp.concatenate(cols, axis=1)
    cv = jnp.stack(consts)[None, :]
    nt16 = f @ V + cv
    nt_ref[...] = nt16.T
    brows = []
    for hh in range(HEAD):
        Bh = (jnp.max(jnp.abs(nt16[:, 2 + hh]))
              + jnp.max(jnp.abs(nt16[:, 6 + hh])))
        brows.append(jnp.broadcast_to(Bh, (1, 128)))
    brows.append(jnp.zeros((4, 128), _f32))
    bv_ref[...] = jnp.concatenate(brows, axis=0)


def _pre2_body(f_ref, Ww_ref, bw_ref, h_ref):
    h_ref[...] = f_ref[...] @ Ww_ref[...] + bw_ref[...]


def _rden_body(parts_ref, rd_ref):
    s = parts_ref[0:N, 0:HEAD] + parts_ref[N:2 * N, 0:HEAD]
    rd_ref[...] = 1.0 / (s + 1e-16)


def _post_body(parts_ref, Wl_ref, bl_ref, o_ref):
    o = parts_ref[0:N, :] + parts_ref[N:2 * N, :]
    o_ref[...] = o @ Wl_ref[...] + bl_ref[...]


def _lp_body(bi_ref, mask_ref, lab_ref, y_ref):
    t = bi_ref[...] * mask_ref[...]
    den = jnp.maximum(jnp.sum(jnp.abs(t), axis=1, keepdims=True), 1e-12)
    y_ref[...] = (t @ lab_ref[...]) / den


def _splat_i(v):
    return jnp.full((16,), v, _i32)


def _edge_scalars(nt_v, srcg, dstg, bvecs):
    a_s = plsc.load_gather(nt_v, [_splat_i(0), srcg])
    b_d = plsc.load_gather(nt_v, [_splat_i(1), dstg])
    x = a_s + b_d
    t = jnp.exp(-2.0 * jnp.abs(x))
    score = jnp.sign(x) * (1.0 - t) / (1.0 + t)
    exs = []
    for hh in range(HEAD):
        p_s = plsc.load_gather(nt_v, [_splat_i(2 + hh), srcg])
        q_d = plsc.load_gather(nt_v, [_splat_i(6 + hh), dstg])
        z = score * p_s + q_d
        alpha = jnp.maximum(z, 0.01 * z)
        exs.append(jnp.exp(alpha - bvecs[hh]))
    return score, exs


_MESH = plsc.VectorSubcoreMesh(core_axis_name="c", subcore_axis_name="s",
                               num_cores=2, num_subcores=16)


@functools.partial(
    pl.kernel,
    out_type=jax.ShapeDtypeStruct((2 * N, 16), _f32),
    mesh=_MESH,
    compiler_params=pltpu.CompilerParams(use_tc_tiling_on_sc=False, needs_layout_passes=False),
    scratch_types=[
        pltpu.VMEM((10, N), _f32),
        pltpu.VMEM((8, 128), _f32),
        pltpu.VMEM((EPW,), _i32),
        pltpu.VMEM((EPW,), _i32),
        pltpu.VMEM((8, BE1), _i32),
        pltpu.VMEM((BE1, 16), _f32),
        pltpu.VMEM_SHARED((N, 16), _f32),
    ],
)
def _sc_sden(nt_hbm, bv_hbm, src_hbm, dst_hbm, out_hbm,
             nt_v, bv_v, src_v, dst_v, didx_v, exb_v, sden_sh):
    cid = lax.axis_index("c")
    sid = lax.axis_index("s")
    wid = cid * 16 + sid

    pltpu.sync_copy(nt_hbm, nt_v)
    pltpu.sync_copy(bv_hbm, bv_v)
    pltpu.sync_copy(src_hbm.at[pl.ds(wid * EPW, EPW)], src_v)
    pltpu.sync_copy(dst_hbm.at[pl.ds(wid * EPW, EPW)], dst_v)

    zero = jnp.zeros((16,), _f32)
    for r in range(BE1):
        exb_v[r, :] = zero
    for tt in range(256 // BE1):
        pltpu.sync_copy(exb_v, sden_sh.at[pl.ds(sid * 256 + tt * BE1, BE1)])
    plsc.subcore_barrier()

    iota = lax.iota(_i32, 16)
    bvecs = [bv_v[hh, pl.ds(0, 16)] for hh in range(HEAD)]

    def batch(j, carry):
        for k in range(GPB1):
            srcg = src_v[pl.ds(j * BE1 + k * 16, 16)]
            dstg = dst_v[pl.ds(j * BE1 + k * 16, 16)]
            didx_v[0, pl.ds(k * 16, 16)] = dstg
            _, exs = _edge_scalars(nt_v, srcg, dstg, bvecs)
            for hh in range(HEAD):
                plsc.store_scatter(exb_v, [iota + k * 16, _splat_i(hh)],
                                   exs[hh])
        pltpu.sync_copy(exb_v, sden_sh.at[didx_v.at[0]], add=True)
        return carry

    lax.fori_loop(0, NB1, batch, 0)
    plsc.subcore_barrier()
    pltpu.sync_copy(sden_sh.at[pl.ds(sid * 256, 256)],
                    out_hbm.at[pl.ds(cid * N + sid * 256, 256)])


@functools.partial(
    pl.kernel,
    out_type=jax.ShapeDtypeStruct((2 * N, HD), _f32),
    mesh=_MESH,
    compiler_params=pltpu.CompilerParams(use_tc_tiling_on_sc=False, needs_layout_passes=False),
    scratch_types=[
        pltpu.VMEM((10, N), _f32),
        pltpu.VMEM((8, 128), _f32),
        pltpu.VMEM((N, HEAD), _f32),
        pltpu.VMEM((EPW,), _i32),
        pltpu.VMEM((EPW,), _i32),
        pltpu.VMEM((8, SB), _i32),
        pltpu.VMEM((BE, DH), _f32),
        pltpu.VMEM((BE, DH), _f32),
        pltpu.VMEM((SB, HD), _f32),
        pltpu.VMEM_SHARED((N, HD), _f32),
        pltpu.SemaphoreType.DMA,
        pltpu.SemaphoreType.DMA,
    ],
)
def _sc_msg(nt_hbm, bv_hbm, rd_hbm, h_hbm, src_hbm, dst_hbm,
            out_hbm, nt_v, bv_v, rd_v, src_v, dst_v, didx_v, rows0_v,
            rows1_v, msg_v, out_sh, sem0, sem1):
    cid = lax.axis_index("c")
    sid = lax.axis_index("s")
    wid = cid * 16 + sid

    pltpu.sync_copy(nt_hbm, nt_v)
    pltpu.sync_copy(bv_hbm, bv_v)
    pltpu.sync_copy(rd_hbm, rd_v)
    pltpu.sync_copy(src_hbm.at[pl.ds(wid * EPW, EPW)], src_v)
    pltpu.sync_copy(dst_hbm.at[pl.ds(wid * EPW, EPW)], dst_v)

    zero = jnp.zeros((16,), _f32)
    for r in range(SB):
        for s4 in range(HD // 16):
            msg_v[r, pl.ds(s4 * 16, 16)] = zero
    for tt in range(256 // SB):
        pltpu.sync_copy(msg_v, out_sh.at[pl.ds(sid * 256 + tt * SB, SB)])
    plsc.subcore_barrier()

    bvecs = [bv_v[hh, pl.ds(0, 16)] for hh in range(HEAD)]

    def compute_batch(j, rows_v, moff):
        for k in range(GPB):
            srcg = src_v[pl.ds(j * BE + k * 16, 16)]
            dstg = dst_v[pl.ds(j * BE + k * 16, 16)]
            didx_v[0, pl.ds(moff + k * 16, 16)] = dstg
            score, exs = _edge_scalars(nt_v, srcg, dstg, bvecs)
            cs = []
            for hh in range(HEAD):
                rd_d = plsc.load_gather(rd_v, [dstg, _splat_i(hh)])
                cs.append(exs[hh] * rd_d * score)
            for e in range(16):
                r = k * 16 + e
                c0 = cs[0][e]
                c1 = cs[1][e]
                c2 = cs[2][e]
                c3 = cs[3][e]
                for s4 in range(HD // 16):
                    m = (c0 * rows_v[r, pl.ds(s4 * 16, 16)]
                         + c1 * rows_v[r, pl.ds(HD + s4 * 16, 16)]
                         + c2 * rows_v[r, pl.ds(2 * HD + s4 * 16, 16)]
                         + c3 * rows_v[r, pl.ds(3 * HD + s4 * 16, 16)])
                    msg_v[moff + r, pl.ds(s4 * 16, 16)] = m

    def gather_rows(j):
        return h_hbm.at[src_v.at[pl.ds(j * BE, BE)]]

    pltpu.async_copy(gather_rows(0), rows0_v, sem0)

    def pair(i, carry):
        ja = 2 * i
        jb = ja + 1
        pltpu.async_copy(gather_rows(jb), rows1_v, sem1)
        pltpu.make_async_copy(gather_rows(ja), rows0_v, sem0).wait()
        compute_batch(ja, rows0_v, 0)
        jn = jnp.minimum(ja + 2, NB - 2)
        pltpu.async_copy(gather_rows(jn), rows0_v, sem0)
        pltpu.make_async_copy(gather_rows(jb), rows1_v, sem1).wait()
        compute_batch(jb, rows1_v, BE)
        pltpu.sync_copy(msg_v, out_sh.at[didx_v.at[0]], add=True)
        return carry

    lax.fori_loop(0, NB // 2, pair, 0)
    pltpu.make_async_copy(gather_rows(0), rows0_v, sem0).wait()
    plsc.subcore_barrier()
    pltpu.sync_copy(out_sh.at[pl.ds(sid * 256, 256)],
                    out_hbm.at[pl.ds(cid * N + sid * 256, 256)])


def kernel(features, labels, bi_adj, adjacency_mask, Wd, bd, Wf, bf, Ww, bw,
           Wa, ba, Wl, bl, edge_index):
    src = edge_index[0]
    dst = edge_index[1]

    nt, bv = pl.pallas_call(
        _pre1_body,
        out_shape=[jax.ShapeDtypeStruct((10, N), _f32),
                   jax.ShapeDtypeStruct((8, 128), _f32)],
    )(features, Wd, bd.reshape(1, DH), Wf, bf.reshape(1, 1), Ww,
      bw.reshape(1, DH), Wa, ba.reshape(1, 1))

    h = pl.pallas_call(
        _pre2_body,
        out_shape=jax.ShapeDtypeStruct((N, DH), _f32),
    )(features, Ww, bw.reshape(1, DH))

    sden_parts = _sc_sden(nt, bv, src, dst)

    rden = pl.pallas_call(
        _rden_body,
        out_shape=jax.ShapeDtypeStruct((N, HEAD), _f32),
    )(sden_parts)

    out_parts = _sc_msg(nt, bv, rden, h, src, dst)

    h_out = pl.pallas_call(
        _post_body,
        out_shape=jax.ShapeDtypeStruct((N, OUT_D), _f32),
    )(out_parts, Wl, bl.reshape(1, OUT_D))

    RB = 256
    y_hat = pl.pallas_call(
        _lp_body,
        grid=(N // RB,),
        in_specs=[pl.BlockSpec((RB, N), lambda i: (i, 0)),
                  pl.BlockSpec((RB, N), lambda i: (i, 0)),
                  pl.BlockSpec((N, 2), lambda i: (0, 0))],
        out_specs=pl.BlockSpec((RB, 2), lambda i: (i, 0)),
        out_shape=jax.ShapeDtypeStruct((N, 2), _f32),
    )(bi_adj, adjacency_mask, labels)

    return h_out, y_hat


# speedup vs baseline: 79.6935x; 1.0229x over previous
import functools
import jax
import jax.numpy as jnp
from jax import lax
from jax.experimental import pallas as pl
from jax.experimental.pallas import tpu as pltpu
from jax.experimental.pallas import tpu_sc as plsc

N = 4096
E = 131072
IN_D = 128
OUT_D = 64
HEAD = 4
HD = 64
DH = OUT_D * HEAD

NW = 32
EPW = E // NW
BE = 32
NB = EPW // BE
GPB = BE // 16
SB = 2 * BE
BE1 = 128
NB1 = EPW // BE1
GPB1 = BE1 // 16

_f32 = jnp.float32
_i32 = jnp.int32


def _pre1_body(f_ref, Wd_ref, bd_ref, Wf_ref, bf_ref, Ww_ref, bw_ref, Wa_ref,
               ba_ref, nt_ref, bv_ref):
    f = f_ref[...]
    Wd = Wd_ref[...]
    Ww = Ww_ref[...]
    bw = bw_ref[...]
    wf1 = Wf_ref[0:DH, :]
    wf2 = Wf_ref[DH:2 * DH, :]
    wf3 = Wf_ref[2 * DH:3 * DH, :]
    w13 = wf1 + wf3
    w23 = wf2 - wf3
    wa1 = Wa_ref[0:HD, :]
    wa2 = Wa_ref[HD:2 * HD, :]

    cols = [Wd @ w13, Wd @ w23]
    consts = [(bd_ref[...] @ w13)[0, 0] + bf_ref[0, 0],
              (bd_ref[...] @ w23)[0, 0]]
    for hh in range(HEAD):
        blk = Ww[:, hh * HD:(hh + 1) * HD]
        bwb = bw[:, hh * HD:(hh + 1) * HD]
        cols.append(blk @ wa1)
        consts.append((bwb @ wa1)[0, 0])
    for hh in range(HEAD):
        blk = Ww[:, hh * HD:(hh + 1) * HD]
        bwb = bw[:, hh * HD:(hh + 1) * HD]
        cols.append(blk @ wa2)
        consts.append((bwb @ wa2)[0, 0] + ba_ref[0, 0])

    V = jnp.concatenate(cols, axis=1)
    cv = jnp.stack(consts)[None, :]
    nt = (f @ V + cv).T
    nt_ref[...] = nt
    brows = []
    for hh in range(HEAD):
        Bh = (jnp.max(jnp.abs(nt[2 + hh, :]))
              + jnp.max(jnp.abs(nt[6 + hh, :])))
        brows.append(jnp.broadcast_to(Bh, (1, 128)))
    brows.append(jnp.zeros((4, 128), _f32))
    bv_ref[...] = jnp.concatenate(brows, axis=0)


def _pre2_body(f_ref, Ww_ref, bw_ref, h_ref):
    h_ref[...] = f_ref[...] @ Ww_ref[...] + bw_ref[...]


def _rden_body(parts_ref, rd_ref):
    s = parts_ref[0:N, 0:HEAD] + parts_ref[N:2 * N, 0:HEAD]
    rd_ref[...] = 1.0 / (s + 1e-16)


def _post_body(parts_ref, Wl_ref, bl_ref, o_ref):
    o = parts_ref[0:N, :] + parts_ref[N:2 * N, :]
    o_ref[...] = o @ Wl_ref[...] + bl_ref[...]


def _lp_body(bi_ref, mask_ref, lab_ref, y_ref):
    t = bi_ref[...] * mask_ref[...]
    den = jnp.maximum(jnp.sum(jnp.abs(t), axis=1, keepdims=True), 1e-12)
    y_ref[...] = (t @ lab_ref[...]) / den


def _splat_i(v):
    return jnp.full((16,), v, _i32)


def _edge_scalars(nt_v, srcg, dstg, bvecs):
    a_s = plsc.load_gather(nt_v, [_splat_i(0), srcg])
    b_d = plsc.load_gather(nt_v, [_splat_i(1), dstg])
    x = a_s + b_d
    t = jnp.exp(-2.0 * jnp.abs(x))
    score = jnp.sign(x) * (1.0 - t) / (1.0 + t)
    exs = []
    for hh in range(HEAD):
        p_s = plsc.load_gather(nt_v, [_splat_i(2 + hh), srcg])
        q_d = plsc.load_gather(nt_v, [_splat_i(6 + hh), dstg])
        z = score * p_s + q_d
        alpha = jnp.maximum(z, 0.01 * z)
        exs.append(jnp.exp(alpha - bvecs[hh]))
    return score, exs


_MESH = plsc.VectorSubcoreMesh(core_axis_name="c", subcore_axis_name="s",
                               num_cores=2, num_subcores=16)


@functools.partial(
    pl.kernel,
    out_type=jax.ShapeDtypeStruct((2 * N, 16), _f32),
    mesh=_MESH,
    compiler_params=pltpu.CompilerParams(use_tc_tiling_on_sc=False, needs_layout_passes=False),
    scratch_types=[
        pltpu.VMEM((10, N), _f32),
        pltpu.VMEM((8, 128), _f32),
        pltpu.VMEM((EPW,), _i32),
        pltpu.VMEM((EPW,), _i32),
        pltpu.VMEM((8, BE1), _i32),
        pltpu.VMEM((BE1, 16), _f32),
        pltpu.VMEM_SHARED((N, 16), _f32),
    ],
)
def _sc_sden(nt_hbm, bv_hbm, ei_hbm, out_hbm,
             nt_v, bv_v, src_v, dst_v, didx_v, exb_v, sden_sh):
    cid = lax.axis_index("c")
    sid = lax.axis_index("s")
    wid = cid * 16 + sid

    pltpu.sync_copy(nt_hbm, nt_v)
    pltpu.sync_copy(bv_hbm, bv_v)
    pltpu.sync_copy(ei_hbm.at[0, pl.ds(wid * EPW, EPW)], src_v)
    pltpu.sync_copy(ei_hbm.at[1, pl.ds(wid * EPW, EPW)], dst_v)

    zero = jnp.zeros((16,), _f32)
    for r in range(BE1):
        exb_v[r, :] = zero
    for tt in range(256 // BE1):
        pltpu.sync_copy(exb_v, sden_sh.at[pl.ds(sid * 256 + tt * BE1, BE1)])
    plsc.subcore_barrier()

    iota = lax.iota(_i32, 16)
    bvecs = [bv_v[hh, pl.ds(0, 16)] for hh in range(HEAD)]

    def batch(j, carry):
        for k in range(GPB1):
            srcg = src_v[pl.ds(j * BE1 + k * 16, 16)]
            dstg = dst_v[pl.ds(j * BE1 + k * 16, 16)]
            didx_v[0, pl.ds(k * 16, 16)] = dstg
            _, exs = _edge_scalars(nt_v, srcg, dstg, bvecs)
            for hh in range(HEAD):
                plsc.store_scatter(exb_v, [iota + k * 16, _splat_i(hh)],
                                   exs[hh])
        pltpu.sync_copy(exb_v, sden_sh.at[didx_v.at[0]], add=True)
        return carry

    lax.fori_loop(0, NB1, batch, 0)
    plsc.subcore_barrier()
    pltpu.sync_copy(sden_sh.at[pl.ds(sid * 256, 256)],
                    out_hbm.at[pl.ds(cid * N + sid * 256, 256)])


@functools.partial(
    pl.kernel,
    out_type=jax.ShapeDtypeStruct((2 * N, HD), _f32),
    mesh=_MESH,
    compiler_params=pltpu.CompilerParams(use_tc_tiling_on_sc=False, needs_layout_passes=False),
    scratch_types=[
        pltpu.VMEM((10, N), _f32),
        pltpu.VMEM((8, 128), _f32),
        pltpu.VMEM((N, HEAD), _f32),
        pltpu.VMEM((EPW,), _i32),
        pltpu.VMEM((EPW,), _i32),
        pltpu.VMEM((8, SB), _i32),
        pltpu.VMEM((BE, DH), _f32),
        pltpu.VMEM((BE, DH), _f32),
        pltpu.VMEM((SB, HD), _f32),
        pltpu.VMEM_SHARED((N, HD), _f32),
        pltpu.SemaphoreType.DMA,
        pltpu.SemaphoreType.DMA,
    ],
)
def _sc_msg(nt_hbm, bv_hbm, rd_hbm, h_hbm, ei_hbm,
            out_hbm, nt_v, bv_v, rd_v, src_v, dst_v, didx_v, rows0_v,
            rows1_v, msg_v, out_sh, sem0, sem1):
    cid = lax.axis_index("c")
    sid = lax.axis_index("s")
    wid = cid * 16 + sid

    pltpu.sync_copy(nt_hbm, nt_v)
    pltpu.sync_copy(bv_hbm, bv_v)
    pltpu.sync_copy(rd_hbm, rd_v)
    pltpu.sync_copy(ei_hbm.at[0, pl.ds(wid * EPW, EPW)], src_v)
    pltpu.sync_copy(ei_hbm.at[1, pl.ds(wid * EPW, EPW)], dst_v)

    zero = jnp.zeros((16,), _f32)
    for r in range(SB):
        for s4 in range(HD // 16):
            msg_v[r, pl.ds(s4 * 16, 16)] = zero
    for tt in range(256 // SB):
        pltpu.sync_copy(msg_v, out_sh.at[pl.ds(sid * 256 + tt * SB, SB)])
    plsc.subcore_barrier()

    bvecs = [bv_v[hh, pl.ds(0, 16)] for hh in range(HEAD)]

    def compute_batch(j, rows_v, moff):
        for k in range(GPB):
            srcg = src_v[pl.ds(j * BE + k * 16, 16)]
            dstg = dst_v[pl.ds(j * BE + k * 16, 16)]
            didx_v[0, pl.ds(moff + k * 16, 16)] = dstg
            score, exs = _edge_scalars(nt_v, srcg, dstg, bvecs)
            cs = []
            for hh in range(HEAD):
                rd_d = plsc.load_gather(rd_v, [dstg, _splat_i(hh)])
                cs.append(exs[hh] * rd_d * score)
            for e in range(16):
                r = k * 16 + e
                c0 = cs[0][e]
                c1 = cs[1][e]
                c2 = cs[2][e]
                c3 = cs[3][e]
                for s4 in range(HD // 16):
                    m = (c0 * rows_v[r, pl.ds(s4 * 16, 16)]
                         + c1 * rows_v[r, pl.ds(HD + s4 * 16, 16)]
                         + c2 * rows_v[r, pl.ds(2 * HD + s4 * 16, 16)]
                         + c3 * rows_v[r, pl.ds(3 * HD + s4 * 16, 16)])
                    msg_v[moff + r, pl.ds(s4 * 16, 16)] = m

    def gather_rows(j):
        return h_hbm.at[src_v.at[pl.ds(j * BE, BE)]]

    pltpu.async_copy(gather_rows(0), rows0_v, sem0)

    def pair(i, carry):
        ja = 2 * i
        jb = ja + 1
        pltpu.async_copy(gather_rows(jb), rows1_v, sem1)
        pltpu.make_async_copy(gather_rows(ja), rows0_v, sem0).wait()
        compute_batch(ja, rows0_v, 0)
        jn = jnp.minimum(ja + 2, NB - 2)
        pltpu.async_copy(gather_rows(jn), rows0_v, sem0)
        pltpu.make_async_copy(gather_rows(jb), rows1_v, sem1).wait()
        compute_batch(jb, rows1_v, BE)
        pltpu.sync_copy(msg_v, out_sh.at[didx_v.at[0]], add=True)
        return carry

    lax.fori_loop(0, NB // 2, pair, 0)
    pltpu.make_async_copy(gather_rows(0), rows0_v, sem0).wait()
    plsc.subcore_barrier()
    pltpu.sync_copy(out_sh.at[pl.ds(sid * 256, 256)],
                    out_hbm.at[pl.ds(cid * N + sid * 256, 256)])


def kernel(features, labels, bi_adj, adjacency_mask, Wd, bd, Wf, bf, Ww, bw,
           Wa, ba, Wl, bl, edge_index):
    nt, bv = pl.pallas_call(
        _pre1_body,
        out_shape=[jax.ShapeDtypeStruct((10, N), _f32),
                   jax.ShapeDtypeStruct((8, 128), _f32)],
    )(features, Wd, bd.reshape(1, DH), Wf, bf.reshape(1, 1), Ww,
      bw.reshape(1, DH), Wa, ba.reshape(1, 1))

    h = pl.pallas_call(
        _pre2_body,
        out_shape=jax.ShapeDtypeStruct((N, DH), _f32),
    )(features, Ww, bw.reshape(1, DH))

    sden_parts = _sc_sden(nt, bv, edge_index)

    rden = pl.pallas_call(
        _rden_body,
        out_shape=jax.ShapeDtypeStruct((N, HEAD), _f32),
    )(sden_parts)

    out_parts = _sc_msg(nt, bv, rden, h, edge_index)

    h_out = pl.pallas_call(
        _post_body,
        out_shape=jax.ShapeDtypeStruct((N, OUT_D), _f32),
    )(out_parts, Wl, bl.reshape(1, OUT_D))

    RB = 256
    y_hat = pl.pallas_call(
        _lp_body,
        grid=(N // RB,),
        in_specs=[pl.BlockSpec((RB, N), lambda i: (i, 0)),
                  pl.BlockSpec((RB, N), lambda i: (i, 0)),
                  pl.BlockSpec((N, 2), lambda i: (0, 0))],
        out_specs=pl.BlockSpec((RB, 2), lambda i: (i, 0)),
        out_shape=jax.ShapeDtypeStruct((N, 2), _f32),
    )(bi_adj, adjacency_mask, labels)

    return h_out, y_hat
